# Initial kernel scaffold; baseline (speedup 1.0000x reference)
import jax
import jax.numpy as jnp
from jax.experimental import pallas as pl


def kernel(query, key, value, rotations):
    raise NotImplementedError("write your pallas kernel here")


# speedup vs baseline: 3.1136x; 3.1136x over previous
import functools

import jax
import jax.numpy as jnp
from jax import lax
from jax.experimental import pallas as pl
from jax.experimental.pallas import tpu as pltpu
from jax.experimental.pallas import tpu_sc as plsc

T = 1024
NH = 4
NBK = 16
NC = NH * NBK
CS = 64
D = 64
DR = 128
NP = 128
NE = NH * T
NROWS = NP * NE


def _stage_a_body(qk_ref, v_ref, rot_ref, pos_ref, qv_ref):
    n = pl.program_id(0)
    qk = qk_ref[0]
    rot = rot_ref[0]
    rotated = jnp.dot(qk, rot, preferred_element_type=jnp.float32)

    iota16 = lax.broadcasted_iota(jnp.int32, (T, NBK), 1)
    r_i = lax.broadcasted_iota(jnp.int32, (T, T), 0)
    c_i = lax.broadcasted_iota(jnp.int32, (T, T), 1)
    lstrict = (r_i > c_i).astype(jnp.float32)

    onehots = []
    ranks = []
    counts = []
    for h in range(NH):
        r = rotated[:, 8 * h:8 * h + 8]
        r16 = jnp.concatenate([r, -r], axis=1)
        m = jnp.max(r16, axis=1, keepdims=True)
        bh = jnp.min(jnp.where(r16 == m, iota16, NBK), axis=1,
                     keepdims=True)
        onehot = (bh == iota16).astype(jnp.float32)
        excl = jnp.dot(lstrict, onehot, preferred_element_type=jnp.float32)
        ranks.append(jnp.sum(onehot * excl, axis=1, keepdims=True))
        counts.append(jnp.sum(onehot, axis=0, keepdims=True))
        onehots.append(onehot)

    cnt64 = jnp.concatenate(counts, axis=1)
    u_r = lax.broadcasted_iota(jnp.int32, (NC, NC), 0)
    u_c = lax.broadcasted_iota(jnp.int32, (NC, NC), 1)
    ustrict = (u_r < u_c).astype(jnp.float32)
    off64 = jnp.dot(cnt64, ustrict, preferred_element_type=jnp.float32)

    base = n * NE
    for h in range(NH):
        off_h = off64[:, NBK * h:NBK * h + NBK]
        posf = ranks[h] + jnp.sum(onehots[h] * off_h, axis=1, keepdims=True)
        posi = posf.astype(jnp.int32) + base
        pos_ref[0, pl.ds(h * T, T), :] = posi

    v = v_ref[0]
    toks_i = lax.broadcasted_iota(jnp.int32, (T, 1), 0)
    v0b = lax.bitcast_convert_type(v[:, 0:1], jnp.int32)
    v0_enc = lax.bitcast_convert_type((v0b & ~1023) | toks_i, jnp.float32)
    qv_ref[0] = jnp.concatenate([qk, v0_enc, v[:, 1:]], axis=1)


_stage_a = pl.pallas_call(
    _stage_a_body,
    grid=(NP,),
    in_specs=[
        pl.BlockSpec((1, T, D), lambda n: (n, 0, 0)),
        pl.BlockSpec((1, T, D), lambda n: (n % 32, 0, 0)),
        pl.BlockSpec((1, D, 32), lambda n: (n // 32, 0, 0)),
    ],
    out_specs=[
        pl.BlockSpec((1, NE, 1), lambda n: (n, 0, 0)),
        pl.BlockSpec((1, T, DR), lambda n: (n, 0, 0)),
    ],
    out_shape=[
        jax.ShapeDtypeStruct((NP, NE, 1), jnp.int32),
        jax.ShapeDtypeStruct((NP, T, DR), jnp.float32),
    ],
)


def _sorted_attention_chunk(cur, prv):
    bq = cur[:, :D]
    kc = jnp.concatenate([cur[:, :D], prv[:, :D]], axis=0)
    nrm = jnp.sqrt(jnp.sum(kc * kc, axis=1, keepdims=True)) + 1e-6
    bk = kc / nrm

    v0 = jnp.concatenate([cur[:, D:D + 1], prv[:, D:D + 1]], axis=0)
    v0b = lax.bitcast_convert_type(v0, jnp.int32)
    kt = (v0b & 1023).astype(jnp.float32)
    qt = kt[:CS]
    v0_clean = lax.bitcast_convert_type(v0b & ~1023, jnp.float32)
    vrest = jnp.concatenate([cur[:, D + 1:], prv[:, D + 1:]], axis=0)
    vv = jnp.concatenate([v0_clean, vrest], axis=1)

    io_q = lax.broadcasted_iota(jnp.int32, (CS, 32), 1).astype(jnp.float32)
    io_k = lax.broadcasted_iota(jnp.int32, (2 * CS, 32), 1).astype(jnp.float32)
    q_hi = (jnp.floor(qt / 32.0) == io_q).astype(jnp.float32)
    q_lo = ((qt - 32.0 * jnp.floor(qt / 32.0)) == io_q).astype(jnp.float32)
    k_hi = (jnp.floor(kt / 32.0) == io_k).astype(jnp.float32)
    k_lo = ((kt - 32.0 * jnp.floor(kt / 32.0)) == io_k).astype(jnp.float32)
    dn = (((1,), (1,)), ((), ()))
    eq_hi = lax.dot_general(q_hi, k_hi, dn, preferred_element_type=jnp.float32)
    eq_lo = lax.dot_general(q_lo, k_lo, dn, preferred_element_type=jnp.float32)
    self_mask = (eq_hi * eq_lo) > 0.5

    dots = lax.dot_general(bq, bk, dn,
                           preferred_element_type=jnp.float32) * 0.125
    dots = jnp.where(self_mask, -1e5, dots)
    mx = jnp.max(dots, axis=1, keepdims=True)
    ex = jnp.exp(dots - mx)
    sm = jnp.sum(ex, axis=1, keepdims=True)
    lg = mx + jnp.log(sm)
    probs = ex / sm
    bo = jnp.dot(probs, vv, preferred_element_type=jnp.float32)
    return bo, lg


def _stage_b_body(sq_ref, so_ref):
    def body(j, carry):
        jp = lax.rem(j + NC - 1, NC)
        cur = sq_ref[0, pl.ds(j * CS, CS), :]
        prv = sq_ref[0, pl.ds(jp * CS, CS), :]
        bo, lg = _sorted_attention_chunk(cur, prv)
        pad = jnp.zeros((CS, DR - D - 1), jnp.float32)
        so_ref[0, pl.ds(j * CS, CS), :] = jnp.concatenate([bo, lg, pad],
                                                          axis=1)
        return carry
    lax.fori_loop(0, NC, body, 0)


_stage_b = pl.pallas_call(
    _stage_b_body,
    grid=(NP,),
    in_specs=[
        pl.BlockSpec((1, NE, DR), lambda n: (n, 0, 0)),
    ],
    out_specs=pl.BlockSpec((1, NE, DR), lambda n: (n, 0, 0)),
    out_shape=jax.ShapeDtypeStruct((NP, NE, DR), jnp.float32),
)


def _stage_d_body(ou_ref, out_ref):
    os_ = []
    lgs = []
    for h in range(NH):
        blk = ou_ref[0, pl.ds(h * T, T), :]
        os_.append(blk[:, :D])
        lgs.append(blk[:, D:D + 1])
    lg = jnp.concatenate(lgs, axis=1)
    m = jnp.max(lg, axis=1, keepdims=True)
    s = jnp.sum(jnp.exp(lg - m), axis=1, keepdims=True)
    acc = jnp.zeros((T, D), jnp.float32)
    for h in range(NH):
        w = jnp.exp(lgs[h] - m) / s
        acc = acc + os_[h] * w
    out_ref[0] = acc


_stage_d = pl.pallas_call(
    _stage_d_body,
    grid=(NP,),
    in_specs=[pl.BlockSpec((1, NE, DR), lambda n: (n, 0, 0))],
    out_specs=pl.BlockSpec((1, T, D), lambda n: (n, 0, 0)),
    out_shape=jax.ShapeDtypeStruct((NP, T, D), jnp.float32),
)


_NWORK = 32
_TILE = 128
_P_PER_W = NP // _NWORK
_TILES_PER_P = NE // _TILE


@functools.lru_cache(maxsize=None)
def _build_sc_kernels():
    mesh = plsc.VectorSubcoreMesh(core_axis_name="c", subcore_axis_name="s")

    @functools.partial(
        pl.kernel,
        mesh=mesh,
        out_type=jax.ShapeDtypeStruct((NROWS, DR), jnp.float32),
        scratch_types=[
            pltpu.VMEM((_TILE,), jnp.int32),
            pltpu.VMEM((_TILE, DR), jnp.float32),
            pltpu.SemaphoreType.DMA,
        ],
    )
    def _sc_scatter(qv_hbm, pos_hbm, oqv_hbm, idx_v, qr, sem):
        wid = lax.axis_index("s") * 2 + lax.axis_index("c")

        def body(t, carry):
            n = wid * _P_PER_W + t // _TILES_PER_P
            e0 = lax.rem(t, _TILES_PER_P) * _TILE
            t0 = lax.rem(e0, T)
            pltpu.sync_copy(pos_hbm.at[n, pl.ds(e0, _TILE)], idx_v)
            pltpu.sync_copy(qv_hbm.at[pl.ds(n * T + t0, _TILE)], qr)
            pltpu.async_copy(qr, oqv_hbm.at[idx_v], sem).wait()
            return carry

        lax.fori_loop(0, _P_PER_W * _TILES_PER_P, body, 0)

    @functools.partial(
        pl.kernel,
        mesh=mesh,
        out_type=jax.ShapeDtypeStruct((NROWS, DR), jnp.float32),
        scratch_types=[
            pltpu.VMEM((_TILE,), jnp.int32),
            pltpu.VMEM((_TILE, DR), jnp.float32),
            pltpu.SemaphoreType.DMA,
        ],
    )
    def _sc_gather(so_hbm, pos_hbm, ou_hbm, idx_v, rows, sem):
        wid = lax.axis_index("s") * 2 + lax.axis_index("c")

        def body(t, carry):
            n = wid * _P_PER_W + t // _TILES_PER_P
            e0 = lax.rem(t, _TILES_PER_P) * _TILE
            pltpu.sync_copy(pos_hbm.at[n, pl.ds(e0, _TILE)], idx_v)
            pltpu.async_copy(so_hbm.at[idx_v], rows, sem).wait()
            pltpu.sync_copy(rows, ou_hbm.at[pl.ds(n * NE + e0, _TILE)])
            return carry

        lax.fori_loop(0, _P_PER_W * _TILES_PER_P, body, 0)

    return _sc_scatter, _sc_gather


def kernel(query, key, value, rotations):
    B, S, H, d = query.shape
    q_r = (query.reshape(B, 4, T, H, d)
           .transpose(1, 0, 3, 2, 4)
           .reshape(NP, T, d))
    v_r = value.transpose(0, 2, 1, 3).reshape(B * H, T, d)
    rot_r = rotations.reshape(4, d, 32)

    sc_scatter, sc_gather = _build_sc_kernels()
    pos3, qv = _stage_a(q_r, v_r, rot_r)
    pos = pos3.reshape(NP, NE)
    sqv = sc_scatter(qv.reshape(NP * T, DR), pos)
    so = _stage_b(sqv.reshape(NP, NE, DR))
    ou = sc_gather(so.reshape(NROWS, DR), pos)
    out = _stage_d(ou.reshape(NP, NE, DR))
    att = (out.reshape(4, B, H, T, d)
           .transpose(1, 2, 0, 3, 4)
           .reshape(B, H, S, d))
    return att


# speedup vs baseline: 4.1691x; 1.3390x over previous
import functools

import jax
import jax.numpy as jnp
from jax import lax
from jax.experimental import pallas as pl
from jax.experimental.pallas import tpu as pltpu
from jax.experimental.pallas import tpu_sc as plsc

T = 1024
NH = 4
NBK = 16
NC = NH * NBK
CS = 64
D = 64
DR = 128
NP = 128
NE = NH * T
NROWS = NP * NE


def _stage_a_body(qk_ref, v_ref, rot_ref, pos_ref, qv_ref):
    n = pl.program_id(0)
    qk = qk_ref[0]
    rot = rot_ref[0]
    rotated = jnp.dot(qk, rot, preferred_element_type=jnp.float32)

    iota16 = lax.broadcasted_iota(jnp.int32, (T, NBK), 1)
    r_i = lax.broadcasted_iota(jnp.int32, (T, T), 0)
    c_i = lax.broadcasted_iota(jnp.int32, (T, T), 1)
    lstrict = (r_i > c_i).astype(jnp.float32)

    onehots = []
    ranks = []
    counts = []
    for h in range(NH):
        r = rotated[:, 8 * h:8 * h + 8]
        r16 = jnp.concatenate([r, -r], axis=1)
        m = jnp.max(r16, axis=1, keepdims=True)
        bh = jnp.min(jnp.where(r16 == m, iota16, NBK), axis=1,
                     keepdims=True)
        onehot = (bh == iota16).astype(jnp.float32)
        excl = jnp.dot(lstrict, onehot, preferred_element_type=jnp.float32)
        ranks.append(jnp.sum(onehot * excl, axis=1, keepdims=True))
        counts.append(jnp.sum(onehot, axis=0, keepdims=True))
        onehots.append(onehot)

    cnt64 = jnp.concatenate(counts, axis=1)
    u_r = lax.broadcasted_iota(jnp.int32, (NC, NC), 0)
    u_c = lax.broadcasted_iota(jnp.int32, (NC, NC), 1)
    ustrict = (u_r < u_c).astype(jnp.float32)
    off64 = jnp.dot(cnt64, ustrict, preferred_element_type=jnp.float32)

    base = n * NE
    for h in range(NH):
        off_h = off64[:, NBK * h:NBK * h + NBK]
        posf = ranks[h] + jnp.sum(onehots[h] * off_h, axis=1, keepdims=True)
        posi = posf.astype(jnp.int32) + base
        pos_ref[0, pl.ds(h * T, T), :] = posi

    v = v_ref[0]
    toks_i = lax.broadcasted_iota(jnp.int32, (T, 1), 0)
    v0b = lax.bitcast_convert_type(v[:, 0:1], jnp.int32)
    v0_enc = lax.bitcast_convert_type((v0b & ~1023) | toks_i, jnp.float32)
    qv_ref[0] = jnp.concatenate([qk, v0_enc, v[:, 1:]], axis=1)


_stage_a = pl.pallas_call(
    _stage_a_body,
    grid=(NP,),
    in_specs=[
        pl.BlockSpec((1, T, D), lambda n: (n, 0, 0)),
        pl.BlockSpec((1, T, D), lambda n: (n % 32, 0, 0)),
        pl.BlockSpec((1, D, 32), lambda n: (n // 32, 0, 0)),
    ],
    out_specs=[
        pl.BlockSpec((1, NE, 1), lambda n: (n, 0, 0)),
        pl.BlockSpec((1, T, DR), lambda n: (n, 0, 0)),
    ],
    out_shape=[
        jax.ShapeDtypeStruct((NP, NE, 1), jnp.int32),
        jax.ShapeDtypeStruct((NP, T, DR), jnp.float32),
    ],
)


_GQ = 256
_GK = _GQ + CS
_NG = NE // _GQ


def _stage_b_body(sq_ref, so_ref):
    dn = (((1,), (1,)), ((), ()))
    blk = sq_ref[0]
    qk = blk[:, :D]
    nrm = jnp.sqrt(jnp.sum(qk * qk, axis=1, keepdims=True)) + 1e-6
    kn = qk / nrm

    v0b = lax.bitcast_convert_type(blk[:, D:D + 1], jnp.int32)
    tok = (v0b & 1023).astype(jnp.float32)
    v0c = lax.bitcast_convert_type(v0b & ~1023, jnp.float32)
    vmat = jnp.concatenate([v0c, blk[:, D + 1:]], axis=1)

    e_r = lax.broadcasted_iota(jnp.int32, (512, 512), 0)
    e_c = lax.broadcasted_iota(jnp.int32, (512, 512), 1)
    eye512 = (e_r == e_c).astype(jnp.float32)
    tokT = jnp.concatenate(
        [lax.dot_general(tok[i * 512:(i + 1) * 512], eye512,
                         (((0,), (0,)), ((), ())),
                         preferred_element_type=jnp.float32,
                         precision=lax.Precision.HIGHEST)
         for i in range(NE // 512)], axis=1)

    rr = lax.broadcasted_iota(jnp.int32, (_GQ, _GK), 0)
    cc = lax.broadcasted_iota(jnp.int32, (_GQ, _GK), 1)
    base = (rr // CS) * CS
    band = (cc >= base) & (cc < base + 2 * CS)

    zpad = jnp.zeros((_GQ, DR - D - 1), jnp.float32)
    for g in range(_NG):
        q0 = g * _GQ
        if g == 0:
            kk = jnp.concatenate([kn[NE - CS:], kn[:_GQ]], axis=0)
            vv = jnp.concatenate([vmat[NE - CS:], vmat[:_GQ]], axis=0)
            ktT = jnp.concatenate([tokT[:, NE - CS:], tokT[:, :_GQ]], axis=1)
        else:
            kk = kn[q0 - CS:q0 + _GQ]
            vv = vmat[q0 - CS:q0 + _GQ]
            ktT = tokT[:, q0 - CS:q0 + _GQ]
        qq = qk[q0:q0 + _GQ]
        qt = tok[q0:q0 + _GQ]
        dots = lax.dot_general(qq, kk, dn,
                               preferred_element_type=jnp.float32) * 0.125
        dots = jnp.where(qt == ktT, -1e5, dots)
        dots = jnp.where(band, dots, -1e30)
        mx = jnp.max(dots, axis=1, keepdims=True)
        ex = jnp.exp(dots - mx)
        sm = jnp.sum(ex, axis=1, keepdims=True)
        lg = mx + jnp.log(sm)
        bo = jnp.dot(ex / sm, vv, preferred_element_type=jnp.float32)
        so_ref[0, q0:q0 + _GQ, :] = jnp.concatenate([bo, lg, zpad], axis=1)


_stage_b = pl.pallas_call(
    _stage_b_body,
    grid=(NP,),
    in_specs=[
        pl.BlockSpec((1, NE, DR), lambda n: (n, 0, 0)),
    ],
    out_specs=pl.BlockSpec((1, NE, DR), lambda n: (n, 0, 0)),
    out_shape=jax.ShapeDtypeStruct((NP, NE, DR), jnp.float32),
)


def _stage_d_body(ou_ref, out_ref):
    os_ = []
    lgs = []
    for h in range(NH):
        blk = ou_ref[0, pl.ds(h * T, T), :]
        os_.append(blk[:, :D])
        lgs.append(blk[:, D:D + 1])
    lg = jnp.concatenate(lgs, axis=1)
    m = jnp.max(lg, axis=1, keepdims=True)
    s = jnp.sum(jnp.exp(lg - m), axis=1, keepdims=True)
    acc = jnp.zeros((T, D), jnp.float32)
    for h in range(NH):
        w = jnp.exp(lgs[h] - m) / s
        acc = acc + os_[h] * w
    out_ref[0] = acc


_stage_d = pl.pallas_call(
    _stage_d_body,
    grid=(NP,),
    in_specs=[pl.BlockSpec((1, NE, DR), lambda n: (n, 0, 0))],
    out_specs=pl.BlockSpec((1, T, D), lambda n: (n, 0, 0)),
    out_shape=jax.ShapeDtypeStruct((NP, T, D), jnp.float32),
)


_NWORK = 32
_TILE = 128
_P_PER_W = NP // _NWORK
_TILES_PER_P = NE // _TILE


@functools.lru_cache(maxsize=None)
def _build_sc_kernels():
    mesh = plsc.VectorSubcoreMesh(core_axis_name="c", subcore_axis_name="s")

    @functools.partial(
        pl.kernel,
        mesh=mesh,
        out_type=jax.ShapeDtypeStruct((NROWS, DR), jnp.float32),
        scratch_types=[
            pltpu.VMEM((_TILE,), jnp.int32),
            pltpu.VMEM((_TILE, DR), jnp.float32),
            pltpu.SemaphoreType.DMA,
        ],
    )
    def _sc_scatter(qv_hbm, pos_hbm, oqv_hbm, idx_v, qr, sem):
        wid = lax.axis_index("s") * 2 + lax.axis_index("c")

        def body(t, carry):
            n = wid * _P_PER_W + t // _TILES_PER_P
            e0 = lax.rem(t, _TILES_PER_P) * _TILE
            t0 = lax.rem(e0, T)
            pltpu.sync_copy(pos_hbm.at[n, pl.ds(e0, _TILE)], idx_v)
            pltpu.sync_copy(qv_hbm.at[pl.ds(n * T + t0, _TILE)], qr)
            pltpu.async_copy(qr, oqv_hbm.at[idx_v], sem).wait()
            return carry

        lax.fori_loop(0, _P_PER_W * _TILES_PER_P, body, 0)

    @functools.partial(
        pl.kernel,
        mesh=mesh,
        out_type=jax.ShapeDtypeStruct((NROWS, DR), jnp.float32),
        scratch_types=[
            pltpu.VMEM((_TILE,), jnp.int32),
            pltpu.VMEM((_TILE, DR), jnp.float32),
            pltpu.SemaphoreType.DMA,
        ],
    )
    def _sc_gather(so_hbm, pos_hbm, ou_hbm, idx_v, rows, sem):
        wid = lax.axis_index("s") * 2 + lax.axis_index("c")

        def body(t, carry):
            n = wid * _P_PER_W + t // _TILES_PER_P
            e0 = lax.rem(t, _TILES_PER_P) * _TILE
            pltpu.sync_copy(pos_hbm.at[n, pl.ds(e0, _TILE)], idx_v)
            pltpu.async_copy(so_hbm.at[idx_v], rows, sem).wait()
            pltpu.sync_copy(rows, ou_hbm.at[pl.ds(n * NE + e0, _TILE)])
            return carry

        lax.fori_loop(0, _P_PER_W * _TILES_PER_P, body, 0)

    return _sc_scatter, _sc_gather


def kernel(query, key, value, rotations):
    B, S, H, d = query.shape
    q_r = (query.reshape(B, 4, T, H, d)
           .transpose(1, 0, 3, 2, 4)
           .reshape(NP, T, d))
    v_r = value.transpose(0, 2, 1, 3).reshape(B * H, T, d)
    rot_r = rotations.reshape(4, d, 32)

    sc_scatter, sc_gather = _build_sc_kernels()
    pos3, qv = _stage_a(q_r, v_r, rot_r)
    pos = pos3.reshape(NP, NE)
    sqv = sc_scatter(qv.reshape(NP * T, DR), pos)
    so = _stage_b(sqv.reshape(NP, NE, DR))
    ou = sc_gather(so.reshape(NROWS, DR), pos)
    out = _stage_d(ou.reshape(NP, NE, DR))
    att = (out.reshape(4, B, H, T, d)
           .transpose(1, 2, 0, 3, 4)
           .reshape(B, H, S, d))
    return att


# speedup vs baseline: 6.8232x; 1.6366x over previous
import functools

import jax
import jax.numpy as jnp
from jax import lax
from jax.experimental import pallas as pl
from jax.experimental.pallas import tpu as pltpu
from jax.experimental.pallas import tpu_sc as plsc

T = 1024
NH = 4
NBK = 16
NC = NH * NBK
CS = 64
D = 64
DR = 128
NP = 128
NE = NH * T
NROWS = NP * NE


def _stage_a_body(qk_ref, v_ref, rot_ref, pos_ref, qv_ref):
    n = pl.program_id(0)
    qk = qk_ref[0]
    rot = rot_ref[0]
    rotated = jnp.dot(qk, rot, preferred_element_type=jnp.float32)

    iota16 = lax.broadcasted_iota(jnp.int32, (T, NBK), 1)
    r_i = lax.broadcasted_iota(jnp.int32, (T, T), 0)
    c_i = lax.broadcasted_iota(jnp.int32, (T, T), 1)
    lstrict = (r_i > c_i).astype(jnp.float32)

    onehots = []
    ranks = []
    counts = []
    for h in range(NH):
        r = rotated[:, 8 * h:8 * h + 8]
        r16 = jnp.concatenate([r, -r], axis=1)
        m = jnp.max(r16, axis=1, keepdims=True)
        bh = jnp.min(jnp.where(r16 == m, iota16, NBK), axis=1,
                     keepdims=True)
        onehot = (bh == iota16).astype(jnp.float32)
        excl = jnp.dot(lstrict, onehot, preferred_element_type=jnp.float32)
        ranks.append(jnp.sum(onehot * excl, axis=1, keepdims=True))
        counts.append(jnp.sum(onehot, axis=0, keepdims=True))
        onehots.append(onehot)

    cnt64 = jnp.concatenate(counts, axis=1)
    u_r = lax.broadcasted_iota(jnp.int32, (NC, NC), 0)
    u_c = lax.broadcasted_iota(jnp.int32, (NC, NC), 1)
    ustrict = (u_r < u_c).astype(jnp.float32)
    off64 = jnp.dot(cnt64, ustrict, preferred_element_type=jnp.float32)

    base = n * NE
    for h in range(NH):
        off_h = off64[:, NBK * h:NBK * h + NBK]
        posf = ranks[h] + jnp.sum(onehots[h] * off_h, axis=1, keepdims=True)
        posi = posf.astype(jnp.int32) + base
        pos_ref[0, pl.ds(h * T, T), :] = posi

    v = v_ref[0]
    toks_i = lax.broadcasted_iota(jnp.int32, (T, 1), 0)
    v0b = lax.bitcast_convert_type(v[:, 0:1], jnp.int32)
    v0_enc = lax.bitcast_convert_type((v0b & ~1023) | toks_i, jnp.float32)
    qv_ref[0] = jnp.concatenate([qk, v0_enc, v[:, 1:]], axis=1)


_stage_a = pl.pallas_call(
    _stage_a_body,
    grid=(NP,),
    in_specs=[
        pl.BlockSpec((1, T, D), lambda n: (n, 0, 0)),
        pl.BlockSpec((1, T, D), lambda n: (n % 32, 0, 0)),
        pl.BlockSpec((1, D, 32), lambda n: (n // 32, 0, 0)),
    ],
    out_specs=[
        pl.BlockSpec((1, NE, 1), lambda n: (n, 0, 0)),
        pl.BlockSpec((1, T, DR), lambda n: (n, 0, 0)),
    ],
    out_shape=[
        jax.ShapeDtypeStruct((NP, NE, 1), jnp.int32),
        jax.ShapeDtypeStruct((NP, T, DR), jnp.float32),
    ],
)


_GQ = 256
_GK = _GQ + CS
_NG = NE // _GQ


def _stage_b_body(sq_ref, so_ref):
    dn = (((1,), (1,)), ((), ()))
    blk = sq_ref[0]
    qk = blk[:, :D]
    nrm = jnp.sqrt(jnp.sum(qk * qk, axis=1, keepdims=True)) + 1e-6
    kn = qk * (1.0 / nrm)

    v0b = lax.bitcast_convert_type(blk[:, D:D + 1], jnp.int32)
    tok = (v0b & 1023).astype(jnp.float32)
    v0c = lax.bitcast_convert_type(v0b & ~1023, jnp.float32)
    vmat = jnp.concatenate([v0c, blk[:, D + 1:],
                            jnp.ones((NE, 1), jnp.float32)], axis=1)

    e_r = lax.broadcasted_iota(jnp.int32, (512, 512), 0)
    e_c = lax.broadcasted_iota(jnp.int32, (512, 512), 1)
    eye512 = (e_r == e_c).astype(jnp.float32)
    tokT = jnp.concatenate(
        [lax.dot_general(tok[i * 512:(i + 1) * 512], eye512,
                         (((0,), (0,)), ((), ())),
                         preferred_element_type=jnp.float32,
                         precision=lax.Precision.HIGHEST)
         for i in range(NE // 512)], axis=1)

    rr = lax.broadcasted_iota(jnp.int32, (_GQ, _GK), 0)
    cc = lax.broadcasted_iota(jnp.int32, (_GQ, _GK), 1)
    base = (rr // CS) * CS
    band = (cc >= base) & (cc < base + 2 * CS)

    zpad = jnp.zeros((_GQ, DR - D - 1), jnp.float32)
    for g in range(_NG):
        q0 = g * _GQ
        if g == 0:
            kk = jnp.concatenate([kn[NE - CS:], kn[:_GQ]], axis=0)
            vv = jnp.concatenate([vmat[NE - CS:], vmat[:_GQ]], axis=0)
            ktT = jnp.concatenate([tokT[:, NE - CS:], tokT[:, :_GQ]], axis=1)
        else:
            kk = kn[q0 - CS:q0 + _GQ]
            vv = vmat[q0 - CS:q0 + _GQ]
            ktT = tokT[:, q0 - CS:q0 + _GQ]
        qq = qk[q0:q0 + _GQ]
        qt = tok[q0:q0 + _GQ]
        dots = lax.dot_general(qq, kk, dn,
                               preferred_element_type=jnp.float32) * 0.125
        valid = band & (qt != ktT)
        ex = jnp.exp(dots) * valid.astype(jnp.float32)
        bo_sm = jnp.dot(ex, vv, preferred_element_type=jnp.float32)
        sm = bo_sm[:, D:D + 1]
        lg = jnp.log(sm)
        bo = bo_sm[:, :D] * (1.0 / sm)
        so_ref[0, q0:q0 + _GQ, :] = jnp.concatenate([bo, lg, zpad], axis=1)


_stage_b = pl.pallas_call(
    _stage_b_body,
    grid=(NP,),
    in_specs=[
        pl.BlockSpec((1, NE, DR), lambda n: (n, 0, 0)),
    ],
    out_specs=pl.BlockSpec((1, NE, DR), lambda n: (n, 0, 0)),
    out_shape=jax.ShapeDtypeStruct((NP, NE, DR), jnp.float32),
)


def _stage_d_body(ou_ref, out_ref):
    os_ = []
    lgs = []
    for h in range(NH):
        blk = ou_ref[0, pl.ds(h * T, T), :]
        os_.append(blk[:, :D])
        lgs.append(blk[:, D:D + 1])
    lg = jnp.concatenate(lgs, axis=1)
    m = jnp.max(lg, axis=1, keepdims=True)
    s = jnp.sum(jnp.exp(lg - m), axis=1, keepdims=True)
    acc = jnp.zeros((T, D), jnp.float32)
    for h in range(NH):
        w = jnp.exp(lgs[h] - m) / s
        acc = acc + os_[h] * w
    out_ref[0] = acc


_stage_d = pl.pallas_call(
    _stage_d_body,
    grid=(NP,),
    in_specs=[pl.BlockSpec((1, NE, DR), lambda n: (n, 0, 0))],
    out_specs=pl.BlockSpec((1, T, D), lambda n: (n, 0, 0)),
    out_shape=jax.ShapeDtypeStruct((NP, T, D), jnp.float32),
)


_NWORK = 32
_TILE = 128
_P_PER_W = NP // _NWORK
_TILES_PER_P = NE // _TILE


@functools.lru_cache(maxsize=None)
def _build_sc_kernels():
    mesh = plsc.VectorSubcoreMesh(core_axis_name="c", subcore_axis_name="s")

    @functools.partial(
        pl.kernel,
        mesh=mesh,
        out_type=jax.ShapeDtypeStruct((NROWS, DR), jnp.float32),
        scratch_types=[
            pltpu.VMEM((_TILE,), jnp.int32),
            pltpu.VMEM((_TILE, DR), jnp.float32),
            pltpu.SemaphoreType.DMA,
        ],
    )
    def _sc_scatter(qv_hbm, pos_hbm, oqv_hbm, idx_v, qr, sem):
        wid = lax.axis_index("s") * 2 + lax.axis_index("c")

        def body(t, carry):
            n = wid * _P_PER_W + t // _TILES_PER_P
            e0 = lax.rem(t, _TILES_PER_P) * _TILE
            t0 = lax.rem(e0, T)
            pltpu.sync_copy(pos_hbm.at[n, pl.ds(e0, _TILE)], idx_v)
            pltpu.sync_copy(qv_hbm.at[pl.ds(n * T + t0, _TILE)], qr)
            pltpu.async_copy(qr, oqv_hbm.at[idx_v], sem).wait()
            return carry

        lax.fori_loop(0, _P_PER_W * _TILES_PER_P, body, 0)

    @functools.partial(
        pl.kernel,
        mesh=mesh,
        out_type=jax.ShapeDtypeStruct((NROWS, DR), jnp.float32),
        scratch_types=[
            pltpu.VMEM((_TILE,), jnp.int32),
            pltpu.VMEM((_TILE, DR), jnp.float32),
            pltpu.SemaphoreType.DMA,
        ],
    )
    def _sc_gather(so_hbm, pos_hbm, ou_hbm, idx_v, rows, sem):
        wid = lax.axis_index("s") * 2 + lax.axis_index("c")

        def body(t, carry):
            n = wid * _P_PER_W + t // _TILES_PER_P
            e0 = lax.rem(t, _TILES_PER_P) * _TILE
            pltpu.sync_copy(pos_hbm.at[n, pl.ds(e0, _TILE)], idx_v)
            pltpu.async_copy(so_hbm.at[idx_v], rows, sem).wait()
            pltpu.sync_copy(rows, ou_hbm.at[pl.ds(n * NE + e0, _TILE)])
            return carry

        lax.fori_loop(0, _P_PER_W * _TILES_PER_P, body, 0)

    return _sc_scatter, _sc_gather


def kernel(query, key, value, rotations):
    B, S, H, d = query.shape
    q_r = (query.reshape(B, 4, T, H, d)
           .transpose(1, 0, 3, 2, 4)
           .reshape(NP, T, d))
    v_r = value.transpose(0, 2, 1, 3).reshape(B * H, T, d)
    rot_r = rotations.reshape(4, d, 32)

    sc_scatter, sc_gather = _build_sc_kernels()
    pos3, qv = _stage_a(q_r, v_r, rot_r)
    pos = pos3.reshape(NP, NE)
    sqv = sc_scatter(qv.reshape(NP * T, DR), pos)
    so = _stage_b(sqv.reshape(NP, NE, DR))
    ou = sc_gather(so.reshape(NROWS, DR), pos)
    out = _stage_d(ou.reshape(NP, NE, DR))
    att = (out.reshape(4, B, H, T, d)
           .transpose(1, 2, 0, 3, 4)
           .reshape(B, H, S, d))
    return att


# speedup vs baseline: 7.5544x; 1.1072x over previous
import functools

import jax
import jax.numpy as jnp
from jax import lax
from jax.experimental import pallas as pl
from jax.experimental.pallas import tpu as pltpu
from jax.experimental.pallas import tpu_sc as plsc

T = 1024
NH = 4
NBK = 16
NC = NH * NBK
CS = 64
D = 64
DR = 128
NP = 128
NE = NH * T
NROWS = NP * NE


def _stage_a_body(qk_ref, v_ref, rot_ref, ltri_ref, pos_ref, qv_ref):
    n = pl.program_id(0)
    qk = qk_ref[0]
    rot = rot_ref[0]
    rotated = jnp.dot(qk, rot, preferred_element_type=jnp.float32)

    iota16 = lax.broadcasted_iota(jnp.int32, (T, NBK), 1)
    onehots = []
    for h in range(NH):
        r = rotated[:, 8 * h:8 * h + 8]
        r16 = jnp.concatenate([r, -r], axis=1)
        m = jnp.max(r16, axis=1, keepdims=True)
        bh = jnp.min(jnp.where(r16 == m, iota16, NBK), axis=1,
                     keepdims=True)
        onehots.append((bh == iota16).astype(jnp.float32))

    o_all = jnp.concatenate(onehots, axis=1)
    e_all = jnp.dot(ltri_ref[...], o_all.astype(jnp.bfloat16),
                    preferred_element_type=jnp.float32)
    cnt64 = jnp.sum(o_all, axis=0, keepdims=True)
    u_r = lax.broadcasted_iota(jnp.int32, (NC, NC), 0)
    u_c = lax.broadcasted_iota(jnp.int32, (NC, NC), 1)
    ustrict = (u_r < u_c).astype(jnp.float32)
    off64 = jnp.dot(cnt64, ustrict, preferred_element_type=jnp.float32,
                    precision=lax.Precision.HIGHEST)
    m_all = e_all + off64

    base = n * NE
    for h in range(NH):
        sl = slice(NBK * h, NBK * h + NBK)
        posf = jnp.sum(onehots[h] * m_all[:, sl], axis=1, keepdims=True)
        posi = posf.astype(jnp.int32) + base
        pos_ref[0, pl.ds(h * T, T), :] = posi

    v = v_ref[0]
    toks_i = lax.broadcasted_iota(jnp.int32, (T, 1), 0)
    v0b = lax.bitcast_convert_type(v[:, 0:1], jnp.int32)
    v0_enc = lax.bitcast_convert_type((v0b & ~1023) | toks_i, jnp.float32)
    qv_ref[0] = jnp.concatenate([qk, v0_enc, v[:, 1:]], axis=1)


_stage_a = pl.pallas_call(
    _stage_a_body,
    grid=(NP,),
    in_specs=[
        pl.BlockSpec((1, T, D), lambda n: (n, 0, 0)),
        pl.BlockSpec((1, T, D), lambda n: (n % 32, 0, 0)),
        pl.BlockSpec((1, D, 32), lambda n: (n // 32, 0, 0)),
        pl.BlockSpec((T, T), lambda n: (0, 0)),
    ],
    out_specs=[
        pl.BlockSpec((1, NE, 1), lambda n: (n, 0, 0)),
        pl.BlockSpec((1, T, DR), lambda n: (n, 0, 0)),
    ],
    out_shape=[
        jax.ShapeDtypeStruct((NP, NE, 1), jnp.int32),
        jax.ShapeDtypeStruct((NP, T, DR), jnp.float32),
    ],
)


_GQ = 256
_GK = _GQ + CS
_NG = NE // _GQ


def _stage_b_body(sq_ref, eye_ref, band_ref, so_ref):
    dn = (((1,), (1,)), ((), ()))
    blk = sq_ref[0]
    qk = blk[:, :D]
    nrm = jnp.sqrt(jnp.sum(qk * qk, axis=1, keepdims=True)) + 1e-6
    kn = qk * (1.0 / nrm)

    v0b = lax.bitcast_convert_type(blk[:, D:D + 1], jnp.int32)
    tok = (v0b & 1023).astype(jnp.float32)
    v0c = lax.bitcast_convert_type(v0b & ~1023, jnp.float32)
    vmat = jnp.concatenate([v0c, blk[:, D + 1:],
                            jnp.ones((NE, 1), jnp.float32)], axis=1)

    eye512 = eye_ref[...]
    tokT = jnp.concatenate(
        [lax.dot_general(tok[i * 512:(i + 1) * 512], eye512,
                         (((0,), (0,)), ((), ())),
                         preferred_element_type=jnp.float32,
                         precision=lax.Precision.HIGHEST)
         for i in range(NE // 512)], axis=1)

    band = band_ref[...]

    zpad = jnp.zeros((_GQ, DR - D - 1), jnp.float32)
    for g in range(_NG):
        q0 = g * _GQ
        if g == 0:
            kk = jnp.concatenate([kn[NE - CS:], kn[:_GQ]], axis=0)
            vv = jnp.concatenate([vmat[NE - CS:], vmat[:_GQ]], axis=0)
            ktT = jnp.concatenate([tokT[:, NE - CS:], tokT[:, :_GQ]], axis=1)
        else:
            kk = kn[q0 - CS:q0 + _GQ]
            vv = vmat[q0 - CS:q0 + _GQ]
            ktT = tokT[:, q0 - CS:q0 + _GQ]
        qq = qk[q0:q0 + _GQ]
        qt = tok[q0:q0 + _GQ]
        dots = lax.dot_general(qq, kk, dn,
                               preferred_element_type=jnp.float32) * 0.125
        valid = band * (qt != ktT).astype(jnp.float32)
        ex = jnp.exp(dots) * valid
        bo_sm = jnp.dot(ex, vv, preferred_element_type=jnp.float32)
        sm = bo_sm[:, D:D + 1]
        lg = jnp.log(sm)
        bo = bo_sm[:, :D] * (1.0 / sm)
        so_ref[0, q0:q0 + _GQ, :] = jnp.concatenate([bo, lg, zpad], axis=1)


_stage_b = pl.pallas_call(
    _stage_b_body,
    grid=(NP,),
    in_specs=[
        pl.BlockSpec((1, NE, DR), lambda n: (n, 0, 0)),
        pl.BlockSpec((512, 512), lambda n: (0, 0)),
        pl.BlockSpec((_GQ, _GK), lambda n: (0, 0)),
    ],
    out_specs=pl.BlockSpec((1, NE, DR), lambda n: (n, 0, 0)),
    out_shape=jax.ShapeDtypeStruct((NP, NE, DR), jnp.float32),
)


def _stage_d_body(ou_ref, out_ref):
    os_ = []
    lgs = []
    for h in range(NH):
        blk = ou_ref[0, pl.ds(h * T, T), :]
        os_.append(blk[:, :D])
        lgs.append(blk[:, D:D + 1])
    lg = jnp.concatenate(lgs, axis=1)
    m = jnp.max(lg, axis=1, keepdims=True)
    s = jnp.sum(jnp.exp(lg - m), axis=1, keepdims=True)
    acc = jnp.zeros((T, D), jnp.float32)
    for h in range(NH):
        w = jnp.exp(lgs[h] - m) / s
        acc = acc + os_[h] * w
    out_ref[0] = acc


_stage_d = pl.pallas_call(
    _stage_d_body,
    grid=(NP,),
    in_specs=[pl.BlockSpec((1, NE, DR), lambda n: (n, 0, 0))],
    out_specs=pl.BlockSpec((1, T, D), lambda n: (n, 0, 0)),
    out_shape=jax.ShapeDtypeStruct((NP, T, D), jnp.float32),
)


_NWORK = 32
_TILE = 128
_P_PER_W = NP // _NWORK
_TILES_PER_P = NE // _TILE


@functools.lru_cache(maxsize=None)
def _build_sc_kernels():
    mesh = plsc.VectorSubcoreMesh(core_axis_name="c", subcore_axis_name="s")

    @functools.partial(
        pl.kernel,
        mesh=mesh,
        out_type=jax.ShapeDtypeStruct((NROWS, DR), jnp.float32),
        scratch_types=[
            pltpu.VMEM((_TILE,), jnp.int32),
            pltpu.VMEM((_TILE, DR), jnp.float32),
            pltpu.SemaphoreType.DMA,
        ],
    )
    def _sc_scatter(qv_hbm, pos_hbm, oqv_hbm, idx_v, qr, sem):
        wid = lax.axis_index("s") * 2 + lax.axis_index("c")

        def body(t, carry):
            n = wid * _P_PER_W + t // _TILES_PER_P
            e0 = lax.rem(t, _TILES_PER_P) * _TILE
            t0 = lax.rem(e0, T)
            pltpu.sync_copy(pos_hbm.at[n, pl.ds(e0, _TILE)], idx_v)
            pltpu.sync_copy(qv_hbm.at[pl.ds(n * T + t0, _TILE)], qr)
            pltpu.async_copy(qr, oqv_hbm.at[idx_v], sem).wait()
            return carry

        lax.fori_loop(0, _P_PER_W * _TILES_PER_P, body, 0)

    @functools.partial(
        pl.kernel,
        mesh=mesh,
        out_type=jax.ShapeDtypeStruct((NROWS, DR), jnp.float32),
        scratch_types=[
            pltpu.VMEM((_TILE,), jnp.int32),
            pltpu.VMEM((_TILE, DR), jnp.float32),
            pltpu.SemaphoreType.DMA,
        ],
    )
    def _sc_gather(so_hbm, pos_hbm, ou_hbm, idx_v, rows, sem):
        wid = lax.axis_index("s") * 2 + lax.axis_index("c")

        def body(t, carry):
            n = wid * _P_PER_W + t // _TILES_PER_P
            e0 = lax.rem(t, _TILES_PER_P) * _TILE
            pltpu.sync_copy(pos_hbm.at[n, pl.ds(e0, _TILE)], idx_v)
            pltpu.async_copy(so_hbm.at[idx_v], rows, sem).wait()
            pltpu.sync_copy(rows, ou_hbm.at[pl.ds(n * NE + e0, _TILE)])
            return carry

        lax.fori_loop(0, _P_PER_W * _TILES_PER_P, body, 0)

    return _sc_scatter, _sc_gather


def kernel(query, key, value, rotations):
    B, S, H, d = query.shape
    q_r = (query.reshape(B, 4, T, H, d)
           .transpose(1, 0, 3, 2, 4)
           .reshape(NP, T, d))
    v_r = value.transpose(0, 2, 1, 3).reshape(B * H, T, d)
    rot_r = rotations.reshape(4, d, 32)

    sc_scatter, sc_gather = _build_sc_kernels()

    r_i = lax.broadcasted_iota(jnp.int32, (T, T), 0)
    c_i = lax.broadcasted_iota(jnp.int32, (T, T), 1)
    ltri = (r_i > c_i).astype(jnp.bfloat16)
    e_r = lax.broadcasted_iota(jnp.int32, (512, 512), 0)
    e_c = lax.broadcasted_iota(jnp.int32, (512, 512), 1)
    eye512 = (e_r == e_c).astype(jnp.float32)
    rr = lax.broadcasted_iota(jnp.int32, (_GQ, _GK), 0)
    cc = lax.broadcasted_iota(jnp.int32, (_GQ, _GK), 1)
    bb = (rr // CS) * CS
    band = ((cc >= bb) & (cc < bb + 2 * CS)).astype(jnp.float32)

    pos3, qv = _stage_a(q_r, v_r, rot_r, ltri)
    pos = pos3.reshape(NP, NE)
    sqv = sc_scatter(qv.reshape(NP * T, DR), pos)
    so = _stage_b(sqv.reshape(NP, NE, DR), eye512, band)
    ou = sc_gather(so.reshape(NROWS, DR), pos)
    out = _stage_d(ou.reshape(NP, NE, DR))
    att = (out.reshape(4, B, H, T, d)
           .transpose(1, 2, 0, 3, 4)
           .reshape(B, H, S, d))
    return att
